# baseline (speedup 1.0000x reference)
import dataclasses
import functools

import jax
import jax.numpy as jnp
from jax import lax
from jax.experimental import pallas as pl
from jax.experimental.pallas import tpu as pltpu
from jax.experimental.pallas import tpu_sc as plsc

N = 10000
E = 320000
D = 128

NC = 2
NS = 16
NW = NC * NS
EPW = E // NW
CH = 200
NCHUNK = EPW // CH
GCH = 192
NFULL = EPW // GCH
TAILE = EPW - NFULL * GCH
RPS = 624
TAIL = N - NS * RPS
TAIL_OFF = NS * RPS

CW = 128

_vmesh = plsc.VectorSubcoreMesh(core_axis_name="c", subcore_axis_name="s")

_no_layout_cp = pltpu.CompilerParams()
if "needs_layout_passes" in pltpu.CompilerParams.__dataclass_fields__:
  _no_layout_cp = dataclasses.replace(_no_layout_cp, needs_layout_passes=False)


def _sc_scatter_sum(values, row_idx, col3_idx, zeros_nd):
  @functools.partial(
      pl.kernel,
      mesh=_vmesh,
      out_type=jax.ShapeDtypeStruct((NC, N, D), jnp.float32),
      scratch_types=[
          pltpu.VMEM((GCH,), jnp.int32),
          pltpu.VMEM((GCH,), jnp.int32),
          pltpu.VMEM((GCH,), jnp.int32),
          pltpu.VMEM((GCH,), jnp.int32),
          pltpu.VMEM((TAILE,), jnp.int32),
          pltpu.VMEM((TAILE,), jnp.int32),
          pltpu.VMEM((GCH, D), jnp.float32),
          pltpu.VMEM((GCH, D), jnp.float32),
          pltpu.VMEM_SHARED((N, D), jnp.float32),
          pltpu.SemaphoreType.DMA,
          pltpu.SemaphoreType.DMA,
          pltpu.SemaphoreType.DMA,
          pltpu.SemaphoreType.DMA,
          pltpu.SemaphoreType.DMA,
          pltpu.SemaphoreType.DMA,
      ],
  )
  def k(vals_hbm, row_hbm, col_hbm, zeros_hbm, out_hbm,
        row_a, row_b, col_a, col_b, row_t, col_t, ga, gb, acc,
        sem_a, sem_b, sem_ca, sem_cb, sem_ra, sem_rb):
    cid = lax.axis_index("c")
    sid = lax.axis_index("s")
    wid = sid * NC + cid
    base = wid * EPW

    pltpu.sync_copy(row_hbm.at[pl.ds(base, GCH)], row_a)
    pltpu.async_copy(vals_hbm.at[row_a], ga, sem_a)
    pltpu.async_copy(col_hbm.at[pl.ds(base, GCH)], col_a, sem_ca)
    pltpu.sync_copy(row_hbm.at[pl.ds(base + GCH, GCH)], row_b)
    pltpu.async_copy(vals_hbm.at[row_b], gb, sem_b)
    pltpu.async_copy(col_hbm.at[pl.ds(base + GCH, GCH)], col_b, sem_cb)

    pltpu.sync_copy(zeros_hbm.at[pl.ds(sid * RPS, RPS)],
                    acc.at[pl.ds(sid * RPS, RPS)])

    @pl.when(sid == 0)
    def _():
      pltpu.sync_copy(zeros_hbm.at[pl.ds(TAIL_OFF, TAIL)],
                      acc.at[pl.ds(TAIL_OFF, TAIL)])

    plsc.subcore_barrier()

    @pl.loop(0, NFULL, step=2)
    def _(k0):
      nxt_a = jnp.where(k0 + 2 < NFULL, (k0 + 2) * GCH, 0)
      nxt_b = jnp.where(k0 + 3 < NFULL, (k0 + 3) * GCH, 0)

      pltpu.make_async_copy(vals_hbm.at[row_a], ga, sem_a).wait()
      ra = pltpu.async_copy(row_hbm.at[pl.ds(base + nxt_a, GCH)], row_a,
                            sem_ra)
      pltpu.make_async_copy(col_hbm.at[pl.ds(base, GCH)], col_a,
                            sem_ca).wait()
      pltpu.sync_copy(ga, acc.at[col_a], add=True)
      ra.wait()
      pltpu.async_copy(vals_hbm.at[row_a], ga, sem_a)
      pltpu.async_copy(col_hbm.at[pl.ds(base + nxt_a, GCH)], col_a, sem_ca)

      pltpu.make_async_copy(vals_hbm.at[row_b], gb, sem_b).wait()
      rb = pltpu.async_copy(row_hbm.at[pl.ds(base + nxt_b, GCH)], row_b,
                            sem_rb)
      pltpu.make_async_copy(col_hbm.at[pl.ds(base, GCH)], col_b,
                            sem_cb).wait()
      pltpu.sync_copy(gb, acc.at[col_b], add=True)
      rb.wait()
      pltpu.async_copy(vals_hbm.at[row_b], gb, sem_b)
      pltpu.async_copy(col_hbm.at[pl.ds(base + nxt_b, GCH)], col_b, sem_cb)

    pltpu.make_async_copy(vals_hbm.at[row_a], ga, sem_a).wait()
    pltpu.make_async_copy(vals_hbm.at[row_b], gb, sem_b).wait()
    pltpu.make_async_copy(col_hbm.at[pl.ds(base, GCH)], col_a, sem_ca).wait()
    pltpu.make_async_copy(col_hbm.at[pl.ds(base, GCH)], col_b, sem_cb).wait()
    pltpu.sync_copy(row_hbm.at[pl.ds(base + NFULL * GCH, TAILE)], row_t)
    pltpu.sync_copy(col_hbm.at[pl.ds(base + NFULL * GCH, TAILE)], col_t)
    pltpu.async_copy(vals_hbm.at[row_t], ga.at[pl.ds(0, TAILE)], sem_a).wait()
    pltpu.sync_copy(ga.at[pl.ds(0, TAILE)], acc.at[col_t], add=True)

    plsc.subcore_barrier()
    pltpu.sync_copy(acc.at[pl.ds(sid * RPS, RPS)],
                    out_hbm.at[cid].at[pl.ds(sid * RPS, RPS)])

    @pl.when(sid == 0)
    def _():
      pltpu.sync_copy(acc.at[pl.ds(TAIL_OFF, TAIL)],
                      out_hbm.at[cid].at[pl.ds(TAIL_OFF, TAIL)])

  return k(values, row_idx, col3_idx, zeros_nd)


NBR = 80


def _sc_count_reg(col_idx, zeros_blk, ident):
  @functools.partial(
      pl.kernel,
      mesh=_vmesh,
      out_type=jax.ShapeDtypeStruct((NC, NBR, 128), jnp.float32),
      compiler_params=_no_layout_cp,
      scratch_types=[
          pltpu.VMEM((EPW,), jnp.int32),
          pltpu.VMEM((NBR, 128), jnp.float32),
          pltpu.VMEM((NBR,), jnp.int32),
          pltpu.VMEM_SHARED((NBR, 128), jnp.float32),
      ],
  )
  def k(col_hbm, zeros_hbm, ident_hbm, out_hbm, col_v, cnt, ident_v, acc):
    cid = lax.axis_index("c")
    sid = lax.axis_index("s")
    wid = sid * NC + cid
    pltpu.sync_copy(col_hbm.at[pl.ds(wid * EPW, EPW)], col_v)
    pltpu.sync_copy(zeros_hbm, cnt)
    pltpu.sync_copy(ident_hbm, ident_v)

    @pl.when(sid == 0)
    def _():
      pltpu.sync_copy(zeros_hbm, acc)

    ones_vec = jnp.ones((16,), jnp.float32)

    @pl.loop(0, EPW // 16)
    def _(i):
      idxv = col_v[pl.ds(i * 16, 16)]
      hi = lax.shift_right_logical(idxv, 7)
      lo = lax.bitwise_and(idxv, 127)
      plsc.addupdate_scatter(cnt, [hi, lo], ones_vec)

    plsc.subcore_barrier()
    pltpu.sync_copy(cnt, acc.at[ident_v], add=True)
    plsc.subcore_barrier()

    @pl.when(sid == 0)
    def _():
      pltpu.sync_copy(acc, out_hbm.at[cid])

  return k(col_idx, zeros_blk, ident)


BR = 1000
GRID = N // BR


def _row_spec():
  return pl.BlockSpec((BR, D), lambda i: (i, 0))


def _w_spec():
  return pl.BlockSpec((D, D), lambda i: (0, 0))


def _b_spec():
  return pl.BlockSpec((1, D), lambda i: (0, 0))


def _cnt_spec():
  return pl.BlockSpec((BR, 1), lambda i: (i, 0))


def _out_nd():
  return jax.ShapeDtypeStruct((N, D), jnp.float32)


def _tc_mm_scale(x, w, c0, c1, bg):
  def kfn(x_ref, w_ref, c0_ref, c1_ref, b_ref, hp_ref, hs_ref):
    h = jnp.dot(x_ref[...], w_ref[...], preferred_element_type=jnp.float32)
    cnt = c0_ref[...] + c1_ref[...]
    dinv = lax.rsqrt(cnt + 1.0)
    hp_ref[...] = h * dinv
    hs_ref[...] = h * (dinv * dinv) + b_ref[...]

  return pl.pallas_call(
      kfn, grid=(GRID,),
      in_specs=[_row_spec(), _w_spec(), _cnt_spec(), _cnt_spec(), _b_spec()],
      out_specs=(_row_spec(), _row_spec()),
      out_shape=(_out_nd(), _out_nd()))(x, w, c0, c1, bg)


def _tc_gcn_combine(p0, p1, c0, c1, hs):
  def kfn(p0_ref, p1_ref, c0_ref, c1_ref, hs_ref, o_ref):
    cnt = c0_ref[...] + c1_ref[...]
    dinv = lax.rsqrt(cnt + 1.0)
    s = (p0_ref[...] + p1_ref[...]) * dinv + hs_ref[...]
    o_ref[...] = jnp.maximum(s, 0.0)

  return pl.pallas_call(
      kfn, grid=(GRID,),
      in_specs=[_row_spec(), _row_spec(),
                _cnt_spec(), _cnt_spec(), _row_spec()],
      out_specs=_row_spec(),
      out_shape=_out_nd())(p0, p1, c0, c1, hs)


def _tc_linear(x, w, b):
  def kfn(x_ref, w_ref, b_ref, o_ref):
    o_ref[...] = jnp.dot(x_ref[...], w_ref[...],
                         preferred_element_type=jnp.float32) + b_ref[...]

  return pl.pallas_call(
      kfn, grid=(GRID,),
      in_specs=[_row_spec(), _w_spec(), _b_spec()],
      out_specs=_row_spec(),
      out_shape=_out_nd())(x, w, b)


def _tc_sage_combine(a0, a1, c0, c1, xr, wl):
  def kfn(a0_ref, a1_ref, c0_ref, c1_ref, xr_ref, wl_ref, o_ref):
    cnt = c0_ref[...] + c1_ref[...]
    agg = (a0_ref[...] + a1_ref[...]) / jnp.maximum(cnt, 1.0)
    s = (jnp.dot(agg, wl_ref[...], preferred_element_type=jnp.float32) +
         xr_ref[...])
    o_ref[...] = jnp.maximum(s, 0.0)

  return pl.pallas_call(
      kfn, grid=(GRID,),
      in_specs=[_row_spec(), _row_spec(), _cnt_spec(), _cnt_spec(),
                _row_spec(), _w_spec()],
      out_specs=_row_spec(),
      out_shape=_out_nd())(a0, a1, c0, c1, xr, wl)


@jax.jit
def kernel(x, edge_index, batch, Wg, bg, Wl1, bl1, Wr1, Wl2, bl2, Wr2):
  del batch
  ei = edge_index.astype(jnp.int32)
  row = ei[0]
  col = ei[1]

  zeros_nd = jnp.zeros((N, D), jnp.float32)
  zeros_blk = jnp.zeros((NBR, 128), jnp.float32)
  ident = jnp.arange(NBR, dtype=jnp.int32)
  bg2 = bg.reshape(1, D)
  bl1_2 = bl1.reshape(1, D)
  bl2_2 = bl2.reshape(1, D)

  counts = _sc_count_reg(col, zeros_blk, ident)
  cflat = counts.reshape(NC, NBR * 128)
  c0 = cflat[0, :N, None]
  c1 = cflat[1, :N, None]
  hp, hs = _tc_mm_scale(x, Wg, c0, c1, bg2)

  p = _sc_scatter_sum(hp, row, col, zeros_nd)
  x1 = _tc_gcn_combine(p[0], p[1], c0, c1, hs)

  a = _sc_scatter_sum(x1, row, col, zeros_nd)
  xr1 = _tc_linear(x1, Wr1, bl1_2)
  x2 = _tc_sage_combine(a[0], a[1], c0, c1, xr1, Wl1)

  b = _sc_scatter_sum(x2, row, col, zeros_nd)
  xr2 = _tc_linear(x2, Wr2, bl2_2)
  x3 = _tc_sage_combine(b[0], b[1], c0, c1, xr2, Wl2)

  return (x1, x2, x3)


# speedup vs baseline: 1.0069x; 1.0069x over previous
import dataclasses
import functools

import jax
import jax.numpy as jnp
from jax import lax
from jax.experimental import pallas as pl
from jax.experimental.pallas import tpu as pltpu
from jax.experimental.pallas import tpu_sc as plsc

N = 10000
E = 320000
D = 128

NC = 2
NS = 16
NW = NC * NS
EPW = E // NW
CH = 200
NCHUNK = EPW // CH
GCH = 192
NFULL = EPW // GCH
TAILE = EPW - NFULL * GCH
RPS = 624
TAIL = N - NS * RPS
TAIL_OFF = NS * RPS

CW = 128

_vmesh = plsc.VectorSubcoreMesh(core_axis_name="c", subcore_axis_name="s")

_no_layout_cp = pltpu.CompilerParams()
if "needs_layout_passes" in pltpu.CompilerParams.__dataclass_fields__:
  _no_layout_cp = dataclasses.replace(_no_layout_cp, needs_layout_passes=False)


def _sc_scatter_sum(values, row_idx, col3_idx, zeros_nd):
  @functools.partial(
      pl.kernel,
      mesh=_vmesh,
      out_type=jax.ShapeDtypeStruct((NC, N, D), jnp.float32),
      scratch_types=[
          pltpu.VMEM((GCH,), jnp.int32),
          pltpu.VMEM((GCH,), jnp.int32),
          pltpu.VMEM((GCH,), jnp.int32),
          pltpu.VMEM((GCH,), jnp.int32),
          pltpu.VMEM((TAILE,), jnp.int32),
          pltpu.VMEM((TAILE,), jnp.int32),
          pltpu.VMEM((GCH, D), jnp.float32),
          pltpu.VMEM((GCH, D), jnp.float32),
          pltpu.VMEM_SHARED((N, D), jnp.float32),
          pltpu.SemaphoreType.DMA,
          pltpu.SemaphoreType.DMA,
          pltpu.SemaphoreType.DMA,
          pltpu.SemaphoreType.DMA,
          pltpu.SemaphoreType.DMA,
          pltpu.SemaphoreType.DMA,
      ],
  )
  def k(vals_hbm, row_hbm, col_hbm, zeros_hbm, out_hbm,
        row_a, row_b, col_a, col_b, row_t, col_t, ga, gb, acc,
        sem_a, sem_b, sem_ca, sem_cb, sem_ra, sem_rb):
    cid = lax.axis_index("c")
    sid = lax.axis_index("s")
    wid = sid * NC + cid
    base = wid * EPW

    pltpu.sync_copy(row_hbm.at[pl.ds(base + NFULL * GCH, TAILE)], row_t)
    pltpu.sync_copy(col_hbm.at[pl.ds(base + NFULL * GCH, TAILE)], col_t)
    pltpu.async_copy(vals_hbm.at[row_t], ga.at[pl.ds(0, TAILE)], sem_a)

    pltpu.sync_copy(zeros_hbm.at[pl.ds(sid * RPS, RPS)],
                    acc.at[pl.ds(sid * RPS, RPS)])

    @pl.when(sid == 0)
    def _():
      pltpu.sync_copy(zeros_hbm.at[pl.ds(TAIL_OFF, TAIL)],
                      acc.at[pl.ds(TAIL_OFF, TAIL)])

    plsc.subcore_barrier()

    pltpu.make_async_copy(vals_hbm.at[row_t], ga.at[pl.ds(0, TAILE)],
                          sem_a).wait()
    pltpu.sync_copy(ga.at[pl.ds(0, TAILE)], acc.at[col_t], add=True)

    pltpu.sync_copy(row_hbm.at[pl.ds(base, GCH)], row_a)
    pltpu.async_copy(vals_hbm.at[row_a], ga, sem_a)
    pltpu.async_copy(col_hbm.at[pl.ds(base, GCH)], col_a, sem_ca)
    pltpu.sync_copy(row_hbm.at[pl.ds(base + GCH, GCH)], row_b)
    pltpu.async_copy(vals_hbm.at[row_b], gb, sem_b)
    pltpu.async_copy(col_hbm.at[pl.ds(base + GCH, GCH)], col_b, sem_cb)

    @pl.loop(0, NFULL, step=2)
    def _(k0):
      pltpu.make_async_copy(vals_hbm.at[row_a], ga, sem_a).wait()

      @pl.when(k0 + 2 < NFULL)
      def _():
        pltpu.async_copy(row_hbm.at[pl.ds(base + (k0 + 2) * GCH, GCH)],
                         row_a, sem_ra)

      pltpu.make_async_copy(col_hbm.at[pl.ds(base, GCH)], col_a,
                            sem_ca).wait()
      pltpu.sync_copy(ga, acc.at[col_a], add=True)

      @pl.when(k0 + 2 < NFULL)
      def _():
        pltpu.make_async_copy(row_hbm.at[pl.ds(base, GCH)], row_a,
                              sem_ra).wait()
        pltpu.async_copy(vals_hbm.at[row_a], ga, sem_a)
        pltpu.async_copy(col_hbm.at[pl.ds(base + (k0 + 2) * GCH, GCH)],
                         col_a, sem_ca)

      pltpu.make_async_copy(vals_hbm.at[row_b], gb, sem_b).wait()

      @pl.when(k0 + 3 < NFULL)
      def _():
        pltpu.async_copy(row_hbm.at[pl.ds(base + (k0 + 3) * GCH, GCH)],
                         row_b, sem_rb)

      pltpu.make_async_copy(col_hbm.at[pl.ds(base, GCH)], col_b,
                            sem_cb).wait()
      pltpu.sync_copy(gb, acc.at[col_b], add=True)

      @pl.when(k0 + 3 < NFULL)
      def _():
        pltpu.make_async_copy(row_hbm.at[pl.ds(base, GCH)], row_b,
                              sem_rb).wait()
        pltpu.async_copy(vals_hbm.at[row_b], gb, sem_b)
        pltpu.async_copy(col_hbm.at[pl.ds(base + (k0 + 3) * GCH, GCH)],
                         col_b, sem_cb)

    plsc.subcore_barrier()
    pltpu.sync_copy(acc.at[pl.ds(sid * RPS, RPS)],
                    out_hbm.at[cid].at[pl.ds(sid * RPS, RPS)])

    @pl.when(sid == 0)
    def _():
      pltpu.sync_copy(acc.at[pl.ds(TAIL_OFF, TAIL)],
                      out_hbm.at[cid].at[pl.ds(TAIL_OFF, TAIL)])

  return k(values, row_idx, col3_idx, zeros_nd)


NBR = 80


def _sc_count_reg(col_idx, zeros_blk, ident):
  @functools.partial(
      pl.kernel,
      mesh=_vmesh,
      out_type=jax.ShapeDtypeStruct((NC, NBR, 128), jnp.float32),
      compiler_params=_no_layout_cp,
      scratch_types=[
          pltpu.VMEM((EPW,), jnp.int32),
          pltpu.VMEM((NBR, 128), jnp.float32),
          pltpu.VMEM((NBR,), jnp.int32),
          pltpu.VMEM_SHARED((NBR, 128), jnp.float32),
      ],
  )
  def k(col_hbm, zeros_hbm, ident_hbm, out_hbm, col_v, cnt, ident_v, acc):
    cid = lax.axis_index("c")
    sid = lax.axis_index("s")
    wid = sid * NC + cid
    pltpu.sync_copy(col_hbm.at[pl.ds(wid * EPW, EPW)], col_v)
    pltpu.sync_copy(zeros_hbm, cnt)
    pltpu.sync_copy(ident_hbm, ident_v)

    @pl.when(sid == 0)
    def _():
      pltpu.sync_copy(zeros_hbm, acc)

    ones_vec = jnp.ones((16,), jnp.float32)

    @pl.loop(0, EPW // 16)
    def _(i):
      idxv = col_v[pl.ds(i * 16, 16)]
      hi = lax.shift_right_logical(idxv, 7)
      lo = lax.bitwise_and(idxv, 127)
      plsc.addupdate_scatter(cnt, [hi, lo], ones_vec)

    plsc.subcore_barrier()
    pltpu.sync_copy(cnt, acc.at[ident_v], add=True)
    plsc.subcore_barrier()

    @pl.when(sid == 0)
    def _():
      pltpu.sync_copy(acc, out_hbm.at[cid])

  return k(col_idx, zeros_blk, ident)


BR = 1000
GRID = N // BR


def _row_spec():
  return pl.BlockSpec((BR, D), lambda i: (i, 0))


def _w_spec():
  return pl.BlockSpec((D, D), lambda i: (0, 0))


def _b_spec():
  return pl.BlockSpec((1, D), lambda i: (0, 0))


def _cnt_spec():
  return pl.BlockSpec((BR, 1), lambda i: (i, 0))


def _out_nd():
  return jax.ShapeDtypeStruct((N, D), jnp.float32)


def _tc_mm_scale(x, w, c0, c1, bg):
  def kfn(x_ref, w_ref, c0_ref, c1_ref, b_ref, hp_ref, hs_ref):
    h = jnp.dot(x_ref[...], w_ref[...], preferred_element_type=jnp.float32)
    cnt = c0_ref[...] + c1_ref[...]
    dinv = lax.rsqrt(cnt + 1.0)
    hp_ref[...] = h * dinv
    hs_ref[...] = h * (dinv * dinv) + b_ref[...]

  return pl.pallas_call(
      kfn, grid=(GRID,),
      in_specs=[_row_spec(), _w_spec(), _cnt_spec(), _cnt_spec(), _b_spec()],
      out_specs=(_row_spec(), _row_spec()),
      out_shape=(_out_nd(), _out_nd()))(x, w, c0, c1, bg)


def _tc_gcn_combine(p0, p1, c0, c1, hs):
  def kfn(p0_ref, p1_ref, c0_ref, c1_ref, hs_ref, o_ref):
    cnt = c0_ref[...] + c1_ref[...]
    dinv = lax.rsqrt(cnt + 1.0)
    s = (p0_ref[...] + p1_ref[...]) * dinv + hs_ref[...]
    o_ref[...] = jnp.maximum(s, 0.0)

  return pl.pallas_call(
      kfn, grid=(GRID,),
      in_specs=[_row_spec(), _row_spec(),
                _cnt_spec(), _cnt_spec(), _row_spec()],
      out_specs=_row_spec(),
      out_shape=_out_nd())(p0, p1, c0, c1, hs)


def _tc_linear(x, w, b):
  def kfn(x_ref, w_ref, b_ref, o_ref):
    o_ref[...] = jnp.dot(x_ref[...], w_ref[...],
                         preferred_element_type=jnp.float32) + b_ref[...]

  return pl.pallas_call(
      kfn, grid=(GRID,),
      in_specs=[_row_spec(), _w_spec(), _b_spec()],
      out_specs=_row_spec(),
      out_shape=_out_nd())(x, w, b)


def _tc_sage_combine(a0, a1, c0, c1, xr, wl):
  def kfn(a0_ref, a1_ref, c0_ref, c1_ref, xr_ref, wl_ref, o_ref):
    cnt = c0_ref[...] + c1_ref[...]
    agg = (a0_ref[...] + a1_ref[...]) / jnp.maximum(cnt, 1.0)
    s = (jnp.dot(agg, wl_ref[...], preferred_element_type=jnp.float32) +
         xr_ref[...])
    o_ref[...] = jnp.maximum(s, 0.0)

  return pl.pallas_call(
      kfn, grid=(GRID,),
      in_specs=[_row_spec(), _row_spec(), _cnt_spec(), _cnt_spec(),
                _row_spec(), _w_spec()],
      out_specs=_row_spec(),
      out_shape=_out_nd())(a0, a1, c0, c1, xr, wl)


@jax.jit
def kernel(x, edge_index, batch, Wg, bg, Wl1, bl1, Wr1, Wl2, bl2, Wr2):
  del batch
  ei = edge_index.astype(jnp.int32)
  row = ei[0]
  col = ei[1]

  zeros_nd = jnp.zeros((N, D), jnp.float32)
  zeros_blk = jnp.zeros((NBR, 128), jnp.float32)
  ident = jnp.arange(NBR, dtype=jnp.int32)
  bg2 = bg.reshape(1, D)
  bl1_2 = bl1.reshape(1, D)
  bl2_2 = bl2.reshape(1, D)

  counts = _sc_count_reg(col, zeros_blk, ident)
  cflat = counts.reshape(NC, NBR * 128)
  c0 = cflat[0, :N, None]
  c1 = cflat[1, :N, None]
  hp, hs = _tc_mm_scale(x, Wg, c0, c1, bg2)

  p = _sc_scatter_sum(hp, row, col, zeros_nd)
  x1 = _tc_gcn_combine(p[0], p[1], c0, c1, hs)

  a = _sc_scatter_sum(x1, row, col, zeros_nd)
  xr1 = _tc_linear(x1, Wr1, bl1_2)
  x2 = _tc_sage_combine(a[0], a[1], c0, c1, xr1, Wl1)

  b = _sc_scatter_sum(x2, row, col, zeros_nd)
  xr2 = _tc_linear(x2, Wr2, bl2_2)
  x3 = _tc_sage_combine(b[0], b[1], c0, c1, xr2, Wl2)

  return (x1, x2, x3)


# speedup vs baseline: 1.0084x; 1.0016x over previous
import dataclasses
import functools

import jax
import jax.numpy as jnp
from jax import lax
from jax.experimental import pallas as pl
from jax.experimental.pallas import tpu as pltpu
from jax.experimental.pallas import tpu_sc as plsc

N = 10000
E = 320000
D = 128

NC = 2
NS = 16
NW = NC * NS
EPW = E // NW
GCH = 192
NFULL = EPW // GCH
TAILE = EPW - NFULL * GCH
RPS = 624
TAIL = N - NS * RPS
TAIL_OFF = NS * RPS

_vmesh = plsc.VectorSubcoreMesh(core_axis_name="c", subcore_axis_name="s")

_no_layout_cp = pltpu.CompilerParams()
if "needs_layout_passes" in pltpu.CompilerParams.__dataclass_fields__:
  _no_layout_cp = dataclasses.replace(_no_layout_cp, needs_layout_passes=False)


def _sc_scatter_sum(values, row_idx, col_idx, zeros_nd):
  @functools.partial(
      pl.kernel,
      mesh=_vmesh,
      out_type=jax.ShapeDtypeStruct((NC, N, D), jnp.float32),
      scratch_types=[
          pltpu.VMEM((GCH,), jnp.int32),
          pltpu.VMEM((GCH,), jnp.int32),
          pltpu.VMEM((GCH,), jnp.int32),
          pltpu.VMEM((GCH,), jnp.int32),
          pltpu.VMEM((TAILE,), jnp.int32),
          pltpu.VMEM((TAILE,), jnp.int32),
          pltpu.VMEM((GCH, D), jnp.float32),
          pltpu.VMEM((GCH, D), jnp.float32),
          pltpu.VMEM_SHARED((N, D), jnp.float32),
          pltpu.SemaphoreType.DMA,
          pltpu.SemaphoreType.DMA,
          pltpu.SemaphoreType.DMA,
          pltpu.SemaphoreType.DMA,
          pltpu.SemaphoreType.DMA,
          pltpu.SemaphoreType.DMA,
      ],
  )
  def k(vals_hbm, row_hbm, col_hbm, zeros_hbm, out_hbm,
        row_a, row_b, col_a, col_b, row_t, col_t, ga, gb, acc,
        sem_a, sem_b, sem_ca, sem_cb, sem_ra, sem_rb):
    cid = lax.axis_index("c")
    sid = lax.axis_index("s")
    wid = sid * NC + cid
    base = wid * EPW

    pltpu.sync_copy(row_hbm.at[pl.ds(base + NFULL * GCH, TAILE)], row_t)
    pltpu.sync_copy(col_hbm.at[pl.ds(base + NFULL * GCH, TAILE)], col_t)
    pltpu.async_copy(vals_hbm.at[row_t], ga.at[pl.ds(0, TAILE)], sem_a)

    pltpu.sync_copy(zeros_hbm.at[pl.ds(sid * RPS, RPS)],
                    acc.at[pl.ds(sid * RPS, RPS)])

    @pl.when(sid == 0)
    def _():
      pltpu.sync_copy(zeros_hbm.at[pl.ds(TAIL_OFF, TAIL)],
                      acc.at[pl.ds(TAIL_OFF, TAIL)])

    plsc.subcore_barrier()

    pltpu.make_async_copy(vals_hbm.at[row_t], ga.at[pl.ds(0, TAILE)],
                          sem_a).wait()
    pltpu.sync_copy(ga.at[pl.ds(0, TAILE)], acc.at[col_t], add=True)

    pltpu.sync_copy(row_hbm.at[pl.ds(base, GCH)], row_a)
    pltpu.async_copy(vals_hbm.at[row_a], ga, sem_a)
    pltpu.async_copy(col_hbm.at[pl.ds(base, GCH)], col_a, sem_ca)
    pltpu.sync_copy(row_hbm.at[pl.ds(base + GCH, GCH)], row_b)
    pltpu.async_copy(vals_hbm.at[row_b], gb, sem_b)
    pltpu.async_copy(col_hbm.at[pl.ds(base + GCH, GCH)], col_b, sem_cb)

    @pl.loop(0, NFULL, step=2)
    def _(k0):
      pltpu.make_async_copy(vals_hbm.at[row_a], ga, sem_a).wait()

      @pl.when(k0 + 2 < NFULL)
      def _():
        pltpu.async_copy(row_hbm.at[pl.ds(base + (k0 + 2) * GCH, GCH)],
                         row_a, sem_ra)

      pltpu.make_async_copy(col_hbm.at[pl.ds(base, GCH)], col_a,
                            sem_ca).wait()
      pltpu.sync_copy(ga, acc.at[col_a], add=True)

      @pl.when(k0 + 2 < NFULL)
      def _():
        pltpu.make_async_copy(row_hbm.at[pl.ds(base, GCH)], row_a,
                              sem_ra).wait()
        pltpu.async_copy(vals_hbm.at[row_a], ga, sem_a)
        pltpu.async_copy(col_hbm.at[pl.ds(base + (k0 + 2) * GCH, GCH)],
                         col_a, sem_ca)

      pltpu.make_async_copy(vals_hbm.at[row_b], gb, sem_b).wait()

      @pl.when(k0 + 3 < NFULL)
      def _():
        pltpu.async_copy(row_hbm.at[pl.ds(base + (k0 + 3) * GCH, GCH)],
                         row_b, sem_rb)

      pltpu.make_async_copy(col_hbm.at[pl.ds(base, GCH)], col_b,
                            sem_cb).wait()
      pltpu.sync_copy(gb, acc.at[col_b], add=True)

      @pl.when(k0 + 3 < NFULL)
      def _():
        pltpu.make_async_copy(row_hbm.at[pl.ds(base, GCH)], row_b,
                              sem_rb).wait()
        pltpu.async_copy(vals_hbm.at[row_b], gb, sem_b)
        pltpu.async_copy(col_hbm.at[pl.ds(base + (k0 + 3) * GCH, GCH)],
                         col_b, sem_cb)

    plsc.subcore_barrier()
    pltpu.sync_copy(acc.at[pl.ds(sid * RPS, RPS)],
                    out_hbm.at[cid].at[pl.ds(sid * RPS, RPS)])

    @pl.when(sid == 0)
    def _():
      pltpu.sync_copy(acc.at[pl.ds(TAIL_OFF, TAIL)],
                      out_hbm.at[cid].at[pl.ds(TAIL_OFF, TAIL)])

  return k(values, row_idx, col_idx, zeros_nd)


NBR = 80


def _sc_count_reg(col_idx, zeros_blk, ident):
  @functools.partial(
      pl.kernel,
      mesh=_vmesh,
      out_type=jax.ShapeDtypeStruct((NC, NBR, 128), jnp.float32),
      compiler_params=_no_layout_cp,
      scratch_types=[
          pltpu.VMEM((EPW,), jnp.int32),
          pltpu.VMEM((NBR, 128), jnp.float32),
          pltpu.VMEM((NBR,), jnp.int32),
          pltpu.VMEM_SHARED((NBR, 128), jnp.float32),
      ],
  )
  def k(col_hbm, zeros_hbm, ident_hbm, out_hbm, col_v, cnt, ident_v, acc):
    cid = lax.axis_index("c")
    sid = lax.axis_index("s")
    wid = sid * NC + cid
    pltpu.sync_copy(col_hbm.at[pl.ds(wid * EPW, EPW)], col_v)
    pltpu.sync_copy(zeros_hbm, cnt)
    pltpu.sync_copy(ident_hbm, ident_v)

    @pl.when(sid == 0)
    def _():
      pltpu.sync_copy(zeros_hbm, acc)

    ones_vec = jnp.ones((16,), jnp.float32)

    @pl.loop(0, EPW // 16)
    def _(i):
      idxv = col_v[pl.ds(i * 16, 16)]
      hi = lax.shift_right_logical(idxv, 7)
      lo = lax.bitwise_and(idxv, 127)
      plsc.addupdate_scatter(cnt, [hi, lo], ones_vec)

    plsc.subcore_barrier()
    pltpu.sync_copy(cnt, acc.at[ident_v], add=True)
    plsc.subcore_barrier()

    @pl.when(sid == 0)
    def _():
      pltpu.sync_copy(acc, out_hbm.at[cid])

  return k(col_idx, zeros_blk, ident)


BR = 1000
GRID = N // BR


def _row_spec():
  return pl.BlockSpec((BR, D), lambda i: (i, 0))


def _w_spec():
  return pl.BlockSpec((D, D), lambda i: (0, 0))


def _b_spec():
  return pl.BlockSpec((1, D), lambda i: (0, 0))


def _cnt_spec():
  return pl.BlockSpec((BR, 1), lambda i: (i, 0))


def _out_nd():
  return jax.ShapeDtypeStruct((N, D), jnp.float32)


def _tc_mm_scale(x, w, c0, c1, bg):
  def kfn(x_ref, w_ref, c0_ref, c1_ref, b_ref, hp_ref, hs_ref):
    h = jnp.dot(x_ref[...], w_ref[...], preferred_element_type=jnp.float32)
    cnt = c0_ref[...] + c1_ref[...]
    dinv = lax.rsqrt(cnt + 1.0)
    hp_ref[...] = h * dinv
    hs_ref[...] = h * (dinv * dinv) + b_ref[...]

  return pl.pallas_call(
      kfn, grid=(GRID,),
      in_specs=[_row_spec(), _w_spec(), _cnt_spec(), _cnt_spec(), _b_spec()],
      out_specs=(_row_spec(), _row_spec()),
      out_shape=(_out_nd(), _out_nd()))(x, w, c0, c1, bg)


def _tc_gcn_combine(p0, p1, c0, c1, hs):
  def kfn(p0_ref, p1_ref, c0_ref, c1_ref, hs_ref, o_ref):
    cnt = c0_ref[...] + c1_ref[...]
    dinv = lax.rsqrt(cnt + 1.0)
    s = (p0_ref[...] + p1_ref[...]) * dinv + hs_ref[...]
    o_ref[...] = jnp.maximum(s, 0.0)

  return pl.pallas_call(
      kfn, grid=(GRID,),
      in_specs=[_row_spec(), _row_spec(),
                _cnt_spec(), _cnt_spec(), _row_spec()],
      out_specs=_row_spec(),
      out_shape=_out_nd())(p0, p1, c0, c1, hs)


def _tc_linear(x, w, b):
  def kfn(x_ref, w_ref, b_ref, o_ref):
    o_ref[...] = jnp.dot(x_ref[...], w_ref[...],
                         preferred_element_type=jnp.float32) + b_ref[...]

  return pl.pallas_call(
      kfn, grid=(GRID,),
      in_specs=[_row_spec(), _w_spec(), _b_spec()],
      out_specs=_row_spec(),
      out_shape=_out_nd())(x, w, b)


def _tc_sage_combine(a0, a1, c0, c1, xr, wl):
  def kfn(a0_ref, a1_ref, c0_ref, c1_ref, xr_ref, wl_ref, o_ref):
    cnt = c0_ref[...] + c1_ref[...]
    agg = (a0_ref[...] + a1_ref[...]) / jnp.maximum(cnt, 1.0)
    s = (jnp.dot(agg, wl_ref[...], preferred_element_type=jnp.float32) +
         xr_ref[...])
    o_ref[...] = jnp.maximum(s, 0.0)

  return pl.pallas_call(
      kfn, grid=(GRID,),
      in_specs=[_row_spec(), _row_spec(), _cnt_spec(), _cnt_spec(),
                _row_spec(), _w_spec()],
      out_specs=_row_spec(),
      out_shape=_out_nd())(a0, a1, c0, c1, xr, wl)


@jax.jit
def kernel(x, edge_index, batch, Wg, bg, Wl1, bl1, Wr1, Wl2, bl2, Wr2):
  del batch
  ei = edge_index.astype(jnp.int32)
  row = ei[0]
  col = ei[1]

  zeros_nd = jnp.zeros((N, D), jnp.float32)
  zeros_blk = jnp.zeros((NBR, 128), jnp.float32)
  ident = jnp.arange(NBR, dtype=jnp.int32)
  bg2 = bg.reshape(1, D)
  bl1_2 = bl1.reshape(1, D)
  bl2_2 = bl2.reshape(1, D)

  counts = _sc_count_reg(col, zeros_blk, ident)
  cflat = counts.reshape(NC, NBR * 128)
  c0 = cflat[0, :N, None]
  c1 = cflat[1, :N, None]
  hp, hs = _tc_mm_scale(x, Wg, c0, c1, bg2)

  p = _sc_scatter_sum(hp, row, col, zeros_nd)
  x1 = _tc_gcn_combine(p[0], p[1], c0, c1, hs)

  a = _sc_scatter_sum(x1, row, col, zeros_nd)
  xr1 = _tc_linear(x1, Wr1, bl1_2)
  x2 = _tc_sage_combine(a[0], a[1], c0, c1, xr1, Wl1)

  b = _sc_scatter_sum(x2, row, col, zeros_nd)
  xr2 = _tc_linear(x2, Wr2, bl2_2)
  x3 = _tc_sage_combine(b[0], b[1], c0, c1, xr2, Wl2)

  return (x1, x2, x3)
